# baseline (speedup 1.0000x reference)
import functools

import jax
import jax.numpy as jnp
from jax import lax
from jax.experimental import pallas as pl
from jax.experimental.pallas import tpu as pltpu
from jax.experimental.pallas import tpu_sc as plsc

_L = 16
_CB = 128


@functools.lru_cache(maxsize=None)
def _make_kernel(V, D, B):
    info = plsc.get_sparse_core_info()
    NC, NS = info.num_cores, info.num_subcores
    NW = NC * NS
    b_per_w = B // NW
    n_subs = b_per_w // _L
    subs_per_cb = _CB // _L
    assert B % (NW * _CB) == 0 and D % 8 == 0 and V % _L == 0

    mesh = plsc.VectorSubcoreMesh(core_axis_name="c", subcore_axis_name="s")

    @functools.partial(
        pl.kernel,
        mesh=mesh,
        compiler_params=pltpu.CompilerParams(use_tc_tiling_on_sc=True,
                                             needs_layout_passes=False),
        out_type=(
            jax.ShapeDtypeStruct((8, D // 8, B), jnp.float32),
            jax.ShapeDtypeStruct((8, D // 8, B), jnp.float32),
        ),
        scratch_types=[
            pltpu.VMEM((b_per_w,), jnp.int32),
            pltpu.VMEM((b_per_w,), jnp.int32),
            pltpu.VMEM((3, 2, 8, D // 8, _CB), jnp.float32),
            pltpu.VMEM((3, 2, 8, D // 8, _CB), jnp.float32),
            pltpu.VMEM((8, D // 8, _CB), jnp.float32),
            pltpu.VMEM((8, D // 8, _CB), jnp.float32),
            pltpu.SemaphoreType.DMA((3,)),
            pltpu.SemaphoreType.DMA((3,)),
        ],
    )
    def k(idx_hbm, h3, c3, o_h, o_c, idx_v, j_v, blk_h, blk_c,
          ob_h, ob_c, sem_h, sem_c):
        wid = lax.axis_index("s") * NC + lax.axis_index("c")
        base = wid * b_per_w
        pltpu.sync_copy(idx_hbm.at[pl.ds(base, b_per_w)], idx_v)
        for t in range(b_per_w // _L):
            sl = pl.ds(t * _L, _L)
            j_v[sl] = lax.bitwise_and(idx_v[sl], _L - 1)

        iota = lax.iota(jnp.int32, _L)
        qbase = lax.shift_right_logical(iota, 3)
        pcol = lax.bitwise_and(iota, 7) * _L

        def fire(tab, blk, sem, s, par):
            vec = idx_v[pl.ds(s * _L, _L)]
            for l in range(_L):
                v = vec[l]
                a = (v // _L) * _L
                pltpu.async_copy(
                    tab.at[:, :, pl.ds(a, _L)],
                    blk.at[par, l // 8, :, :, pl.ds((l % 8) * _L, _L)],
                    sem.at[par])

        def drain(out, ob, sem, par):
            for _ in range(2):
                pltpu.make_async_copy(
                    out.at[:, :, pl.ds(0, _CB)], ob, sem.at[par]).wait()

        def extract(blk, ob, s, par):
            jvec = j_v[pl.ds(s * _L, _L)]
            parvec = jnp.full((_L,), par, jnp.int32)
            cvec = pcol + jvec
            so = lax.rem(s, subs_per_cb) * _L

            def kd_body(kd, _):
                r = lax.shift_right_logical(kd, 3)
                d8 = lax.bitwise_and(kd, 7)
                rvec = jnp.full((_L,), r, jnp.int32)
                dvec = jnp.full((_L,), d8, jnp.int32)
                vals = plsc.load_gather(
                    blk, [parvec, qbase, rvec, dvec, cvec])
                ob[r, d8, pl.ds(so, _L)] = vals
                return 0

            lax.fori_loop(0, D, kd_body, 0)

        def writeback(s):
            cb = lax.div(s, subs_per_cb)
            o = base + cb * _CB
            pltpu.sync_copy(ob_h, o_h.at[:, :, pl.ds(o, _CB)])
            pltpu.sync_copy(ob_c, o_c.at[:, :, pl.ds(o, _CB)])

        fire(h3, blk_h, sem_h, 0, 0)
        fire(c3, blk_c, sem_c, 0, 0)
        fire(h3, blk_h, sem_h, 1, 1)
        fire(c3, blk_c, sem_c, 1, 1)

        def sub_body(s, _):
            par = lax.rem(s, 3)
            npar = lax.rem(s + 2, 3)

            @pl.when(s + 2 < n_subs)
            def _():
                fire(h3, blk_h, sem_h, s + 2, npar)
                fire(c3, blk_c, sem_c, s + 2, npar)
            drain(o_h, ob_h, sem_h, par)
            extract(blk_h, ob_h, s, par)
            drain(o_c, ob_c, sem_c, par)
            extract(blk_c, ob_c, s, par)

            @pl.when(lax.bitwise_and(s, subs_per_cb - 1) == subs_per_cb - 1)
            def _():
                writeback(s)

            return 0

        lax.fori_loop(0, n_subs - 1, sub_body, 0)
        last = n_subs - 1
        lpar = (n_subs - 1) % 3
        drain(o_h, ob_h, sem_h, lpar)
        extract(blk_h, ob_h, last, lpar)
        drain(o_c, ob_c, sem_c, lpar)
        extract(blk_c, ob_c, last, lpar)
        writeback(last)

    return k


def kernel(stock_id, emb_h, emb_c):
    idx = stock_id.reshape(-1).astype(jnp.int32)
    B = idx.shape[0]
    V, D = emb_h.shape
    h3 = emb_h.T.reshape(8, D // 8, V)
    c3 = emb_c.T.reshape(8, D // 8, V)
    o_h, o_c = _make_kernel(V, D, B)(idx, h3, c3)
    return (o_h.reshape(D, B).T, o_c.reshape(D, B).T)


# speedup vs baseline: 1.0139x; 1.0139x over previous
import functools

import jax
import jax.numpy as jnp
from jax import lax
from jax.experimental import pallas as pl
from jax.experimental.pallas import tpu as pltpu
from jax.experimental.pallas import tpu_sc as plsc

_L = 16
_CB = 128


@functools.lru_cache(maxsize=None)
def _make_kernel(V, D, B):
    info = plsc.get_sparse_core_info()
    NC, NS = info.num_cores, info.num_subcores
    NW = NC * NS
    b_per_w = B // NW
    n_subs = b_per_w // _L
    subs_per_cb = _CB // _L
    assert B % (NW * _CB) == 0 and D % 8 == 0 and V % _L == 0

    mesh = plsc.VectorSubcoreMesh(core_axis_name="c", subcore_axis_name="s")

    @functools.partial(
        pl.kernel,
        mesh=mesh,
        compiler_params=pltpu.CompilerParams(use_tc_tiling_on_sc=True,
                                             needs_layout_passes=False),
        out_type=(
            jax.ShapeDtypeStruct((8, D // 8, B), jnp.float32),
            jax.ShapeDtypeStruct((8, D // 8, B), jnp.float32),
        ),
        scratch_types=[
            pltpu.VMEM((b_per_w,), jnp.int32),
            pltpu.VMEM((b_per_w,), jnp.int32),
            pltpu.VMEM((2, 2, 8, D // 8, _CB), jnp.float32),
            pltpu.VMEM((2, 2, 8, D // 8, _CB), jnp.float32),
            pltpu.VMEM((2, 8, D // 8, _CB), jnp.float32),
            pltpu.VMEM((2, 8, D // 8, _CB), jnp.float32),
            pltpu.SemaphoreType.DMA((2,)),
            pltpu.SemaphoreType.DMA((2,)),
            pltpu.SemaphoreType.DMA((2,)),
            pltpu.SemaphoreType.DMA((2,)),
        ],
    )
    def k(idx_hbm, h3, c3, o_h, o_c, idx_v, j_v, blk_h, blk_c,
          ob_h, ob_c, sem_h, sem_c, wb_h, wb_c):
        wid = lax.axis_index("s") * NC + lax.axis_index("c")
        base = wid * b_per_w
        pltpu.sync_copy(idx_hbm.at[pl.ds(base, b_per_w)], idx_v)
        for t in range(b_per_w // _L):
            sl = pl.ds(t * _L, _L)
            j_v[sl] = lax.bitwise_and(idx_v[sl], _L - 1)

        iota = lax.iota(jnp.int32, _L)
        qbase = lax.shift_right_logical(iota, 3)
        pcol = lax.bitwise_and(iota, 7) * _L

        def fire(tab, blk, sem, s, par):
            vec = idx_v[pl.ds(s * _L, _L)]
            for l in range(_L):
                v = vec[l]
                a = (v // _L) * _L
                pltpu.async_copy(
                    tab.at[:, :, pl.ds(a, _L)],
                    blk.at[par, l // 8, :, :, pl.ds((l % 8) * _L, _L)],
                    sem.at[par])

        def drain(out, ob, sem, par):
            for _ in range(2):
                pltpu.make_async_copy(
                    out.at[:, :, pl.ds(0, _CB)], ob.at[0], sem.at[par]).wait()

        def extract(blk, ob, s, par, cpar):
            jvec = j_v[pl.ds(s * _L, _L)]
            parvec = jnp.full((_L,), par, jnp.int32)
            cvec = pcol + jvec
            so = lax.rem(s, subs_per_cb) * _L

            def kd_body(kd, _):
                r = lax.shift_right_logical(kd, 3)
                d8 = lax.bitwise_and(kd, 7)
                rvec = jnp.full((_L,), r, jnp.int32)
                dvec = jnp.full((_L,), d8, jnp.int32)
                vals = plsc.load_gather(
                    blk, [parvec, qbase, rvec, dvec, cvec])
                ob[cpar, r, d8, pl.ds(so, _L)] = vals
                return 0

            lax.fori_loop(0, D, kd_body, 0)

        def wb_wait(par):
            pltpu.make_async_copy(
                o_h.at[:, :, pl.ds(0, _CB)], ob_h.at[0], wb_h.at[par]).wait()
            pltpu.make_async_copy(
                o_c.at[:, :, pl.ds(0, _CB)], ob_c.at[0], wb_c.at[par]).wait()

        def writeback(s, cpar):
            cb = lax.div(s, subs_per_cb)
            o = base + cb * _CB

            @pl.when(cb >= 2)
            def _():
                wb_wait(cpar)

            pltpu.async_copy(
                ob_h.at[cpar], o_h.at[:, :, pl.ds(o, _CB)], wb_h.at[cpar])
            pltpu.async_copy(
                ob_c.at[cpar], o_c.at[:, :, pl.ds(o, _CB)], wb_c.at[cpar])

        fire(h3, blk_h, sem_h, 0, 0)
        fire(c3, blk_c, sem_c, 0, 0)

        def sub_body(s, _):
            par = lax.bitwise_and(s, 1)
            npar = lax.bitwise_and(s + 1, 1)
            cpar = lax.bitwise_and(lax.div(s, subs_per_cb), 1)
            fire(h3, blk_h, sem_h, s + 1, npar)
            fire(c3, blk_c, sem_c, s + 1, npar)
            drain(o_h, ob_h, sem_h, par)
            extract(blk_h, ob_h, s, par, cpar)
            drain(o_c, ob_c, sem_c, par)
            extract(blk_c, ob_c, s, par, cpar)

            @pl.when(lax.bitwise_and(s, subs_per_cb - 1) == subs_per_cb - 1)
            def _():
                writeback(s, cpar)

            return 0

        lax.fori_loop(0, n_subs - 1, sub_body, 0)
        last = n_subs - 1
        lpar = (n_subs - 1) % 2
        lcpar = ((n_subs - 1) // subs_per_cb) % 2
        drain(o_h, ob_h, sem_h, lpar)
        extract(blk_h, ob_h, last, lpar, lcpar)
        drain(o_c, ob_c, sem_c, lpar)
        extract(blk_c, ob_c, last, lpar, lcpar)
        writeback(last, lcpar)
        wb_wait(0)
        wb_wait(1)

    return k


def kernel(stock_id, emb_h, emb_c):
    idx = stock_id.reshape(-1).astype(jnp.int32)
    B = idx.shape[0]
    V, D = emb_h.shape
    h3 = emb_h.T.reshape(8, D // 8, V)
    c3 = emb_c.T.reshape(8, D // 8, V)
    o_h, o_c = _make_kernel(V, D, B)(idx, h3, c3)
    return (o_h.reshape(D, B).T, o_c.reshape(D, B).T)
